# baseline (speedup 1.0000x reference)
import functools

import jax
import jax.numpy as jnp
from jax import lax
from jax.experimental import pallas as pl
from jax.experimental.pallas import tpu as pltpu
from jax.experimental.pallas import tpu_sc as plsc

N_NODES = 10000
N_EDGES = 320000
FEAT = 128

_NCORE = 2
_NSUB = 16
_K = 80
_NIB = 10
_NOUTER = 25
_NB = 4
_EDGES_PER_TILE = N_EDGES // (_NSUB)
_N_PAD = 10112
_ROWS_PER_TILE = _N_PAD // _NSUB

_sc_mesh = plsc.VectorSubcoreMesh(core_axis_name="c", subcore_axis_name="s")


@functools.partial(
    pl.kernel,
    mesh=_sc_mesh,
    out_type=jax.ShapeDtypeStruct((_NCORE, _N_PAD, FEAT), jnp.float32),
    scratch_types=[
        pltpu.VMEM((2, _NIB, _K), jnp.int32),
        pltpu.VMEM((2, _NIB, _K), jnp.int32),
        pltpu.VMEM((_NB, _K, FEAT), jnp.float32),
        pltpu.VMEM_SHARED((_N_PAD, FEAT), jnp.float32),
        pltpu.SemaphoreType.DMA((_NB,)),
        pltpu.SemaphoreType.DMA((_NB,)),
        pltpu.SemaphoreType.DMA,
    ],
)
def _sc_segment_sum(y_hbm, src_hbm, dst_hbm, out_hbm,
                    src_v, dst_v, rows_v, acc_sh, sem_g, sem_s, sem_i):
    c = lax.axis_index("c")
    s = lax.axis_index("s")
    nchunk = _NOUTER * _NIB

    zvec = jnp.zeros((16,), jnp.float32)

    def _zero_body(i, _):
        r = i // (FEAT // 16)
        q = (i % (FEAT // 16)) * 16
        rows_v[0, r, pl.ds(q, 16)] = zvec
        return _

    lax.fori_loop(0, _K * (FEAT // 16), _zero_body, None)

    base = s * _ROWS_PER_TILE
    for j in range(_ROWS_PER_TILE // _K):
        pltpu.sync_copy(rows_v.at[0], acc_sh.at[pl.ds(base + j * _K, _K)])
    _zrem = _ROWS_PER_TILE % _K
    if _zrem:
        pltpu.sync_copy(
            rows_v.at[0, pl.ds(0, _zrem)],
            acc_sh.at[pl.ds(base + (_ROWS_PER_TILE // _K) * _K, _zrem)])
    plsc.subcore_barrier()

    def _start_gather(j):
        slot = (j // _NIB) % 2
        r = j % _NIB
        pltpu.async_copy(y_hbm.at[src_v.at[slot, r]], rows_v.at[j % _NB],
                         sem_g.at[j % _NB])

    def _wait_gather(j):
        pltpu.make_async_copy(y_hbm.at[src_v.at[0, 0]], rows_v.at[j % _NB],
                              sem_g.at[j % _NB]).wait()

    def _start_scatter(j):
        slot = (j // _NIB) % 2
        r = j % _NIB
        pltpu.async_copy(rows_v.at[j % _NB], acc_sh.at[dst_v.at[slot, r]],
                         sem_s.at[j % _NB], add=True)

    def _wait_scatter(b):
        pltpu.make_async_copy(rows_v.at[b], acc_sh.at[dst_v.at[0, 0]],
                              sem_s.at[b]).wait()

    pltpu.sync_copy(src_hbm.at[c, s, 0], src_v.at[0])
    pltpu.sync_copy(dst_hbm.at[c, s, 0], dst_v.at[0])
    pltpu.sync_copy(src_hbm.at[c, s, 1], src_v.at[1])
    pltpu.sync_copy(dst_hbm.at[c, s, 1], dst_v.at[1])
    _start_gather(0)
    _start_gather(1)

    def _chunk_body(j, _):
        ib = j // _NIB
        r = j % _NIB
        pf = jnp.logical_and(ib >= 1, ib + 1 < _NOUTER)

        @pl.when(jnp.logical_and(r == 2, pf))
        def _():
            pltpu.async_copy(src_hbm.at[c, s, ib + 1],
                             src_v.at[(ib + 1) % 2], sem_i)
            pltpu.async_copy(dst_hbm.at[c, s, ib + 1],
                             dst_v.at[(ib + 1) % 2], sem_i)

        @pl.when(j >= _NB - 2)
        def _():
            _wait_scatter((j + 2) % _NB)

        @pl.when(jnp.logical_and(r == _NIB - 2, pf))
        def _():
            pltpu.make_async_copy(src_hbm.at[c, s, 0], src_v.at[0],
                                  sem_i).wait()
            pltpu.make_async_copy(dst_hbm.at[c, s, 0], dst_v.at[0],
                                  sem_i).wait()

        @pl.when(j + 2 < nchunk)
        def _():
            _start_gather(j + 2)

        _wait_gather(j)
        _start_scatter(j)
        return _

    lax.fori_loop(0, nchunk, _chunk_body, None)
    for t in range(_NB - 2):
        _wait_scatter((nchunk - (_NB - 2) + t) % _NB)
    plsc.subcore_barrier()

    pltpu.sync_copy(acc_sh.at[pl.ds(base, _ROWS_PER_TILE)],
                    out_hbm.at[c, pl.ds(base, _ROWS_PER_TILE)])


_BLK = 2000
_NBLK = N_NODES // _BLK


def _mm_body(x_ref, w_ref, o_ref):
    o_ref[...] = jnp.dot(x_ref[0], w_ref[0],
                         preferred_element_type=jnp.float32)


def _relu_mm_body(a_ref, w_ref, o_ref):
    h = jnp.maximum(a_ref[0], 0.0)
    o_ref[...] = jnp.dot(h, w_ref[0], preferred_element_type=jnp.float32)


def _final_body(px_ref, nx_ref, agg_ref, o_ref):
    o_ref[:, 0:FEAT] = px_ref[...]
    o_ref[:, FEAT:2 * FEAT] = jnp.maximum(agg_ref[0], 0.0)
    o_ref[:, 2 * FEAT:3 * FEAT] = nx_ref[...]
    o_ref[:, 3 * FEAT:4 * FEAT] = jnp.maximum(agg_ref[1], 0.0)


def _mm_stacked(x_all, w_all, body):
    return pl.pallas_call(
        body,
        grid=(_NCORE, _NBLK),
        in_specs=[
            pl.BlockSpec((1, _BLK, FEAT), lambda c, i: (c, i, 0)),
            pl.BlockSpec((1, FEAT, FEAT), lambda c, i: (c, 0, 0)),
        ],
        out_specs=pl.BlockSpec((_BLK, FEAT), lambda c, i: (c * _NBLK + i, 0)),
        out_shape=jax.ShapeDtypeStruct((_NCORE * N_NODES, FEAT), jnp.float32),
    )(x_all, w_all)


def _final_concat(pos_x, neg_x, agg):
    return pl.pallas_call(
        _final_body,
        grid=(_NBLK,),
        in_specs=[
            pl.BlockSpec((_BLK, FEAT), lambda i: (i, 0)),
            pl.BlockSpec((_BLK, FEAT), lambda i: (i, 0)),
            pl.BlockSpec((_NCORE, _BLK, FEAT), lambda i: (0, i, 0)),
        ],
        out_specs=pl.BlockSpec((_BLK, 4 * FEAT), lambda i: (i, 0)),
        out_shape=jax.ShapeDtypeStruct((N_NODES, 4 * FEAT), jnp.float32),
    )(pos_x, neg_x, agg)


def kernel(pos_x, pos_edge_index, neg_x, neg_edge_index,
           pos_W0, pos_W1, pos_W2, neg_W0, neg_W1, neg_W2):
    x_all = jnp.stack([pos_x, neg_x])
    w0 = jnp.stack([pos_W0, neg_W0])
    w1 = jnp.stack([pos_W1, neg_W1])
    w2 = jnp.stack([pos_W2, neg_W2])

    src_all = jnp.stack([pos_edge_index[0],
                         neg_edge_index[0] + N_NODES])
    src_all = src_all.reshape(_NCORE, _NSUB, _NOUTER, _NIB, _K)
    dst_all = jnp.stack([pos_edge_index[1], neg_edge_index[1]])
    dst_all = dst_all.reshape(_NCORE, _NSUB, _NOUTER, _NIB, _K)

    y = _mm_stacked(x_all, w0, _mm_body)
    agg = _sc_segment_sum(y, src_all, dst_all)
    y = _mm_stacked(agg, w1, _relu_mm_body)
    agg = _sc_segment_sum(y, src_all, dst_all)
    y = _mm_stacked(agg, w2, _relu_mm_body)
    agg = _sc_segment_sum(y, src_all, dst_all)
    return _final_concat(pos_x, neg_x, agg)


# speedup vs baseline: 1.0816x; 1.0816x over previous
import functools

import jax
import jax.numpy as jnp
from jax import lax
from jax.experimental import pallas as pl
from jax.experimental.pallas import tpu as pltpu
from jax.experimental.pallas import tpu_sc as plsc

N_NODES = 10000
N_EDGES = 320000
FEAT = 128

_NCORE = 2
_NSUB = 16
_K = 80
_NIB = 25
_NOUTER = 10
_NB = 3
_N_PAD = 10112
_ROWS_PER_TILE = _N_PAD // _NSUB

_sc_mesh = plsc.VectorSubcoreMesh(core_axis_name="c", subcore_axis_name="s")


@functools.partial(
    pl.kernel,
    mesh=_sc_mesh,
    out_type=jax.ShapeDtypeStruct((_NCORE, _N_PAD, FEAT), jnp.float32),
    scratch_types=[
        pltpu.VMEM((2, _NIB, _K), jnp.int32),
        pltpu.VMEM((2, _NIB, _K), jnp.int32),
        pltpu.VMEM((_NB, _K, FEAT), jnp.float32),
        pltpu.VMEM_SHARED((_N_PAD, FEAT), jnp.float32),
        pltpu.SemaphoreType.DMA((_NB,)),
        pltpu.SemaphoreType.DMA((_NB,)),
        pltpu.SemaphoreType.DMA,
    ],
)
def _sc_segment_sum(y_hbm, ps_hbm, pd_hbm, ns_hbm, nd_hbm, out_hbm,
                    src_v, dst_v, rows_v, acc_sh, sem_g, sem_s, sem_i):
    c = lax.axis_index("c")
    s = lax.axis_index("s")
    nchunk = _NOUTER * _NIB

    zvec = jnp.zeros((16,), jnp.float32)

    def _zero_body(i, _):
        r = i // (FEAT // 16)
        q = (i % (FEAT // 16)) * 16
        rows_v[0, r, pl.ds(q, 16)] = zvec
        return _

    lax.fori_loop(0, _K * (FEAT // 16), _zero_body, None)

    base = s * _ROWS_PER_TILE
    for j in range(_ROWS_PER_TILE // _K):
        pltpu.sync_copy(rows_v.at[0], acc_sh.at[pl.ds(base + j * _K, _K)])
    _zrem = _ROWS_PER_TILE % _K
    if _zrem:
        pltpu.sync_copy(
            rows_v.at[0, pl.ds(0, _zrem)],
            acc_sh.at[pl.ds(base + (_ROWS_PER_TILE // _K) * _K, _zrem)])
    plsc.subcore_barrier()

    def _load_idx(ib, slot, sync):
        for e_hbm, e_v in ((ps_hbm, src_v), (pd_hbm, dst_v)):
            @pl.when(c == 0)
            def _():
                cp = pltpu.async_copy(e_hbm.at[s, ib], e_v.at[slot], sem_i)
                if sync:
                    cp.wait()
        for e_hbm, e_v in ((ns_hbm, src_v), (nd_hbm, dst_v)):
            @pl.when(c == 1)
            def _():
                cp = pltpu.async_copy(e_hbm.at[s, ib], e_v.at[slot], sem_i)
                if sync:
                    cp.wait()

    def _drain_idx():
        for e_v in (src_v, dst_v):
            pltpu.make_async_copy(ps_hbm.at[0, 0], e_v.at[0], sem_i).wait()

    def _start_gather(j):
        slot = (j // _NIB) % 2
        r = j % _NIB
        pltpu.async_copy(y_hbm.at[c].at[src_v.at[slot, r]], rows_v.at[j % _NB],
                         sem_g.at[j % _NB])

    def _wait_gather(j):
        pltpu.make_async_copy(y_hbm.at[c].at[src_v.at[0, 0]],
                              rows_v.at[j % _NB], sem_g.at[j % _NB]).wait()

    def _start_scatter(j):
        slot = (j // _NIB) % 2
        r = j % _NIB
        pltpu.async_copy(rows_v.at[j % _NB], acc_sh.at[dst_v.at[slot, r]],
                         sem_s.at[j % _NB], add=True)

    def _wait_scatter(b):
        pltpu.make_async_copy(rows_v.at[b], acc_sh.at[dst_v.at[0, 0]],
                              sem_s.at[b]).wait()

    _load_idx(0, 0, True)
    _load_idx(1, 1, True)
    _start_gather(0)
    _start_gather(1)

    def _chunk_body(j, _):
        ib = j // _NIB
        r = j % _NIB
        pf = jnp.logical_and(ib >= 1, ib + 1 < _NOUTER)

        @pl.when(jnp.logical_and(r == 2, pf))
        def _():
            _load_idx(ib + 1, (ib + 1) % 2, False)

        @pl.when(j >= _NB - 2)
        def _():
            _wait_scatter((j + 2) % _NB)

        @pl.when(jnp.logical_and(r == _NIB - 2, pf))
        def _():
            _drain_idx()

        @pl.when(j + 2 < nchunk)
        def _():
            _start_gather(j + 2)

        _wait_gather(j)
        _start_scatter(j)
        return _

    lax.fori_loop(0, nchunk, _chunk_body, None)
    for t in range(_NB - 2):
        _wait_scatter((nchunk - (_NB - 2) + t) % _NB)
    plsc.subcore_barrier()

    pltpu.sync_copy(acc_sh.at[pl.ds(base, _ROWS_PER_TILE)],
                    out_hbm.at[c, pl.ds(base, _ROWS_PER_TILE)])


_BLK = 2000
_NBLK = N_NODES // _BLK


def _mm_body(px_ref, nx_ref, wp_ref, wn_ref, o_ref):
    o_ref[0] = jnp.dot(px_ref[...], wp_ref[...],
                       preferred_element_type=jnp.float32)
    o_ref[1] = jnp.dot(nx_ref[...], wn_ref[...],
                       preferred_element_type=jnp.float32)


def _relu_mm_body(a_ref, wp_ref, wn_ref, o_ref):
    o_ref[0] = jnp.dot(jnp.maximum(a_ref[0], 0.0), wp_ref[...],
                       preferred_element_type=jnp.float32)
    o_ref[1] = jnp.dot(jnp.maximum(a_ref[1], 0.0), wn_ref[...],
                       preferred_element_type=jnp.float32)


def _final_body(px_ref, nx_ref, agg_ref, o_ref):
    o_ref[:, 0:FEAT] = px_ref[...]
    o_ref[:, FEAT:2 * FEAT] = jnp.maximum(agg_ref[0], 0.0)
    o_ref[:, 2 * FEAT:3 * FEAT] = nx_ref[...]
    o_ref[:, 3 * FEAT:4 * FEAT] = jnp.maximum(agg_ref[1], 0.0)


_x_spec = pl.BlockSpec((_BLK, FEAT), lambda i: (i, 0))
_w_spec = pl.BlockSpec((FEAT, FEAT), lambda i: (0, 0))
_pair_spec = pl.BlockSpec((_NCORE, _BLK, FEAT), lambda i: (0, i, 0))
_y_type = jax.ShapeDtypeStruct((_NCORE, N_NODES, FEAT), jnp.float32)


def _mm0(pos_x, neg_x, wp, wn):
    return pl.pallas_call(
        _mm_body, grid=(_NBLK,),
        in_specs=[_x_spec, _x_spec, _w_spec, _w_spec],
        out_specs=_pair_spec, out_shape=_y_type,
    )(pos_x, neg_x, wp, wn)


def _relu_mm(agg, wp, wn):
    return pl.pallas_call(
        _relu_mm_body, grid=(_NBLK,),
        in_specs=[_pair_spec, _w_spec, _w_spec],
        out_specs=_pair_spec, out_shape=_y_type,
    )(agg, wp, wn)


def _final_concat(pos_x, neg_x, agg):
    return pl.pallas_call(
        _final_body, grid=(_NBLK,),
        in_specs=[_x_spec, _x_spec, _pair_spec],
        out_specs=pl.BlockSpec((_BLK, 4 * FEAT), lambda i: (i, 0)),
        out_shape=jax.ShapeDtypeStruct((N_NODES, 4 * FEAT), jnp.float32),
    )(pos_x, neg_x, agg)


def kernel(pos_x, pos_edge_index, neg_x, neg_edge_index,
           pos_W0, pos_W1, pos_W2, neg_W0, neg_W1, neg_W2):
    eshape = (_NSUB, _NOUTER, _NIB, _K)
    ps = pos_edge_index[0].reshape(eshape)
    pd = pos_edge_index[1].reshape(eshape)
    ns = neg_edge_index[0].reshape(eshape)
    nd = neg_edge_index[1].reshape(eshape)

    y = _mm0(pos_x, neg_x, pos_W0, neg_W0)
    agg = _sc_segment_sum(y, ps, pd, ns, nd)
    y = _relu_mm(agg, pos_W1, neg_W1)
    agg = _sc_segment_sum(y, ps, pd, ns, nd)
    y = _relu_mm(agg, pos_W2, neg_W2)
    agg = _sc_segment_sum(y, ps, pd, ns, nd)
    return _final_concat(pos_x, neg_x, agg)


# speedup vs baseline: 1.1198x; 1.0353x over previous
import functools

import jax
import jax.numpy as jnp
from jax import lax
from jax.experimental import pallas as pl
from jax.experimental.pallas import tpu as pltpu
from jax.experimental.pallas import tpu_sc as plsc

N_NODES = 10000
N_EDGES = 320000
FEAT = 128

_NCORE = 2
_NSUB = 16
_K = 80
_NIB = 25
_NOUTER = 10
_NB = 3
_N_PAD = 10112
_ROWS_PER_TILE = _N_PAD // _NSUB

_sc_mesh = plsc.VectorSubcoreMesh(core_axis_name="c", subcore_axis_name="s")


@functools.partial(
    pl.kernel,
    mesh=_sc_mesh,
    out_type=jax.ShapeDtypeStruct((_NCORE, _N_PAD, FEAT), jnp.float32),
    scratch_types=[
        pltpu.VMEM((2, _NIB, _K), jnp.int32),
        pltpu.VMEM((2, _NIB, _K), jnp.int32),
        pltpu.VMEM((_NB, _K, FEAT), jnp.float32),
        pltpu.VMEM_SHARED((_N_PAD, FEAT), jnp.float32),
        pltpu.SemaphoreType.DMA((_NB,)),
        pltpu.SemaphoreType.DMA((_NB,)),
        pltpu.SemaphoreType.DMA,
    ],
)
def _sc_segment_sum(y_hbm, pe_hbm, ne_hbm, out_hbm,
                    src_v, dst_v, rows_v, acc_sh, sem_g, sem_s, sem_i):
    c = lax.axis_index("c")
    s = lax.axis_index("s")
    nchunk = _NOUTER * _NIB

    zvec = jnp.zeros((16,), jnp.float32)

    def _zero_body(i, _):
        r = i // (FEAT // 16)
        q = (i % (FEAT // 16)) * 16
        rows_v[0, r, pl.ds(q, 16)] = zvec
        return _

    lax.fori_loop(0, _K * (FEAT // 16), _zero_body, None)

    base = s * _ROWS_PER_TILE
    for j in range(_ROWS_PER_TILE // _K):
        pltpu.sync_copy(rows_v.at[0], acc_sh.at[pl.ds(base + j * _K, _K)])
    _zrem = _ROWS_PER_TILE % _K
    if _zrem:
        pltpu.sync_copy(
            rows_v.at[0, pl.ds(0, _zrem)],
            acc_sh.at[pl.ds(base + (_ROWS_PER_TILE // _K) * _K, _zrem)])
    plsc.subcore_barrier()

    def _load_idx(ib, slot, sync):
        for e_hbm, when in ((pe_hbm, c == 0), (ne_hbm, c == 1)):
            for kind, e_v in ((0, src_v), (1, dst_v)):
                @pl.when(when)
                def _():
                    cp = pltpu.async_copy(e_hbm.at[kind, s, ib],
                                          e_v.at[slot], sem_i)
                    if sync:
                        cp.wait()

    def _drain_idx():
        for e_v in (src_v, dst_v):
            pltpu.make_async_copy(pe_hbm.at[0, 0, 0], e_v.at[0], sem_i).wait()

    def _start_gather(j):
        slot = (j // _NIB) % 2
        r = j % _NIB
        pltpu.async_copy(y_hbm.at[c].at[src_v.at[slot, r]], rows_v.at[j % _NB],
                         sem_g.at[j % _NB])

    def _wait_gather(j):
        pltpu.make_async_copy(y_hbm.at[c].at[src_v.at[0, 0]],
                              rows_v.at[j % _NB], sem_g.at[j % _NB]).wait()

    def _start_scatter(j):
        slot = (j // _NIB) % 2
        r = j % _NIB
        pltpu.async_copy(rows_v.at[j % _NB], acc_sh.at[dst_v.at[slot, r]],
                         sem_s.at[j % _NB], add=True)

    def _wait_scatter(b):
        pltpu.make_async_copy(rows_v.at[b], acc_sh.at[dst_v.at[0, 0]],
                              sem_s.at[b]).wait()

    _load_idx(0, 0, True)
    _load_idx(1, 1, True)
    _start_gather(0)
    _start_gather(1)

    def _chunk_body(j, _):
        ib = j // _NIB
        r = j % _NIB
        pf = jnp.logical_and(ib >= 1, ib + 1 < _NOUTER)

        @pl.when(jnp.logical_and(r == 2, pf))
        def _():
            _load_idx(ib + 1, (ib + 1) % 2, False)

        @pl.when(j >= _NB - 2)
        def _():
            _wait_scatter((j + 2) % _NB)

        @pl.when(jnp.logical_and(r == _NIB - 2, pf))
        def _():
            _drain_idx()

        @pl.when(j + 2 < nchunk)
        def _():
            _start_gather(j + 2)

        _wait_gather(j)
        _start_scatter(j)
        return _

    lax.fori_loop(0, nchunk, _chunk_body, None)
    for t in range(_NB - 2):
        _wait_scatter((nchunk - (_NB - 2) + t) % _NB)
    plsc.subcore_barrier()

    pltpu.sync_copy(acc_sh.at[pl.ds(base, _ROWS_PER_TILE)],
                    out_hbm.at[c, pl.ds(base, _ROWS_PER_TILE)])


_BLK = 2000
_NBLK = N_NODES // _BLK


def _mm_body(px_ref, nx_ref, wp_ref, wn_ref, o_ref):
    o_ref[0] = jnp.dot(px_ref[...], wp_ref[...],
                       preferred_element_type=jnp.float32)
    o_ref[1] = jnp.dot(nx_ref[...], wn_ref[...],
                       preferred_element_type=jnp.float32)


def _relu_mm_body(a_ref, wp_ref, wn_ref, o_ref):
    o_ref[0] = jnp.dot(jnp.maximum(a_ref[0], 0.0), wp_ref[...],
                       preferred_element_type=jnp.float32)
    o_ref[1] = jnp.dot(jnp.maximum(a_ref[1], 0.0), wn_ref[...],
                       preferred_element_type=jnp.float32)


def _final_body(px_ref, nx_ref, agg_ref, o_ref):
    o_ref[:, 0:FEAT] = px_ref[...]
    o_ref[:, FEAT:2 * FEAT] = jnp.maximum(agg_ref[0], 0.0)
    o_ref[:, 2 * FEAT:3 * FEAT] = nx_ref[...]
    o_ref[:, 3 * FEAT:4 * FEAT] = jnp.maximum(agg_ref[1], 0.0)


_x_spec = pl.BlockSpec((_BLK, FEAT), lambda i: (i, 0))
_w_spec = pl.BlockSpec((FEAT, FEAT), lambda i: (0, 0))
_pair_spec = pl.BlockSpec((_NCORE, _BLK, FEAT), lambda i: (0, i, 0))
_y_type = jax.ShapeDtypeStruct((_NCORE, N_NODES, FEAT), jnp.float32)


def _mm0(pos_x, neg_x, wp, wn):
    return pl.pallas_call(
        _mm_body, grid=(_NBLK,),
        in_specs=[_x_spec, _x_spec, _w_spec, _w_spec],
        out_specs=_pair_spec, out_shape=_y_type,
    )(pos_x, neg_x, wp, wn)


def _relu_mm(agg, wp, wn):
    return pl.pallas_call(
        _relu_mm_body, grid=(_NBLK,),
        in_specs=[_pair_spec, _w_spec, _w_spec],
        out_specs=_pair_spec, out_shape=_y_type,
    )(agg, wp, wn)


def _final_concat(pos_x, neg_x, agg):
    return pl.pallas_call(
        _final_body, grid=(_NBLK,),
        in_specs=[_x_spec, _x_spec, _pair_spec],
        out_specs=pl.BlockSpec((_BLK, 4 * FEAT), lambda i: (i, 0)),
        out_shape=jax.ShapeDtypeStruct((N_NODES, 4 * FEAT), jnp.float32),
    )(pos_x, neg_x, agg)


def kernel(pos_x, pos_edge_index, neg_x, neg_edge_index,
           pos_W0, pos_W1, pos_W2, neg_W0, neg_W1, neg_W2):
    eshape = (2, _NSUB, _NOUTER, _NIB, _K)
    pe = pos_edge_index.reshape(eshape)
    ne = neg_edge_index.reshape(eshape)

    y = _mm0(pos_x, neg_x, pos_W0, neg_W0)
    agg = _sc_segment_sum(y, pe, ne)
    y = _relu_mm(agg, pos_W1, neg_W1)
    agg = _sc_segment_sum(y, pe, ne)
    y = _relu_mm(agg, pos_W2, neg_W2)
    agg = _sc_segment_sum(y, pe, ne)
    return _final_concat(pos_x, neg_x, agg)
